# Initial kernel scaffold; baseline (speedup 1.0000x reference)
import functools

import jax
import jax.numpy as jnp
from jax import lax
from jax.experimental import pallas as pl
from jax.experimental.pallas import tpu as pltpu
from jax.experimental.pallas import tpu_sc as plsc

_info = plsc.get_sparse_core_info()
_NC, _NS = _info.num_cores, _info.num_subcores
_NW = _NC * _NS


def _gather_kernel(n_rows, chunk, idx_hbm, table_hbm, out_hbm,
                   idx_v, rows_v, sem):
    per_w = n_rows // _NW
    n_ch = per_w // chunk
    wid = lax.axis_index("s") * _NC + lax.axis_index("c")
    base = wid * per_w

    def body(i, carry):
        off = base + i * chunk
        pltpu.sync_copy(idx_hbm.at[pl.ds(off, chunk)], idx_v)
        pltpu.async_copy(table_hbm.at[idx_v], rows_v, sem).wait()
        pltpu.sync_copy(rows_v, out_hbm.at[pl.ds(off, chunk)])
        return carry

    lax.fori_loop(0, n_ch, body, 0)


def kernel(word_indexes, W):
    B, L = word_indexes.shape
    V, D = W.shape
    n = B * L
    idx = word_indexes.reshape(n).astype(jnp.int32)

    chunk = 1024
    assert n % (_NW * chunk) == 0

    mesh = plsc.VectorSubcoreMesh(core_axis_name="c", subcore_axis_name="s")
    k = pl.kernel(
        functools.partial(_gather_kernel, n, chunk),
        mesh=mesh,
        out_type=jax.ShapeDtypeStruct((n, D), jnp.float32),
        scratch_types=[
            pltpu.VMEM((chunk,), jnp.int32),
            pltpu.VMEM((chunk, D), jnp.float32),
            pltpu.SemaphoreType.DMA,
        ],
    )
    out = k(idx, W)
    return out.reshape(B, L, D)


# speedup vs baseline: 1.0937x; 1.0937x over previous
import functools

import jax
import jax.numpy as jnp
from jax import lax
from jax.experimental import pallas as pl
from jax.experimental.pallas import tpu as pltpu
from jax.experimental.pallas import tpu_sc as plsc

_info = plsc.get_sparse_core_info()
_NC, _NS = _info.num_cores, _info.num_subcores
_NW = _NC * _NS


def _gather_kernel(n_rows, chunk, idx_hbm, table_hbm, out_hbm,
                   idx_v, rows_v, sem):
    per_w = n_rows // _NW
    n_ch = per_w // chunk
    wid = lax.axis_index("s") * _NC + lax.axis_index("c")
    base = wid * per_w

    def body(i, carry):
        off = base + i * chunk
        pltpu.sync_copy(idx_hbm.at[pl.ds(off, chunk)], idx_v)
        pltpu.async_copy(table_hbm.at[idx_v], rows_v, sem).wait()
        pltpu.sync_copy(rows_v, out_hbm.at[pl.ds(off, chunk)])
        return carry

    lax.fori_loop(0, n_ch, body, 0)


def kernel(word_indexes, W):
    B, L = word_indexes.shape
    V, D = W.shape
    n = B * L
    idx = word_indexes.reshape(n).astype(jnp.int32)

    chunk = 1024
    assert n % (_NW * chunk) == 0

    mesh = plsc.VectorSubcoreMesh(core_axis_name="c", subcore_axis_name="s")
    k = pl.kernel(
        functools.partial(_gather_kernel, n, chunk),
        mesh=mesh,
        out_type=jax.ShapeDtypeStruct((n, D), jnp.float32),
        scratch_types=[
            pltpu.VMEM((chunk,), jnp.int32),
            pltpu.VMEM((chunk, D), jnp.float32),
            pltpu.SemaphoreType.DMA,
        ],
        compiler_params=pltpu.CompilerParams(use_tc_tiling_on_sc=False),
    )
    out = k(idx, W)
    return out.reshape(B, L, D)


# speedup vs baseline: 1.1132x; 1.0178x over previous
import functools

import jax
import jax.numpy as jnp
from jax import lax
from jax.experimental import pallas as pl
from jax.experimental.pallas import tpu as pltpu
from jax.experimental.pallas import tpu_sc as plsc

_info = plsc.get_sparse_core_info()
_NC, _NS = _info.num_cores, _info.num_subcores
_NW = _NC * _NS

_CHUNK = 512
_NBUF = 5


def _gather_kernel(n_rows, idx_hbm, table_hbm, out_hbm, idx_v, *scratch):
    rows = scratch[:_NBUF]
    gsem = scratch[_NBUF:2 * _NBUF]
    ssem = scratch[2 * _NBUF:3 * _NBUF]

    per_w = n_rows // _NW
    n_ch = per_w // _CHUNK
    n_grp = n_ch // _NBUF
    wid = lax.axis_index("s") * _NC + lax.axis_index("c")
    base = wid * per_w

    pltpu.sync_copy(idx_hbm.at[pl.ds(base, per_w)], idx_v)

    def gather_start(i, b):
        pltpu.async_copy(
            table_hbm.at[idx_v.at[pl.ds(i * _CHUNK, _CHUNK)]], rows[b], gsem[b])

    def gather_wait(b):
        pltpu.make_async_copy(
            table_hbm.at[idx_v.at[pl.ds(0, _CHUNK)]], rows[b], gsem[b]).wait()

    def store_start(i, b):
        pltpu.async_copy(
            rows[b], out_hbm.at[pl.ds(base + i * _CHUNK, _CHUNK)], ssem[b])

    def store_wait(b):
        pltpu.make_async_copy(
            rows[b], out_hbm.at[pl.ds(base, _CHUNK)], ssem[b]).wait()

    for b in range(_NBUF):
        gather_start(b, b)

    def grp(g, carry):
        for b in range(_NBUF):
            i = g * _NBUF + b
            gather_wait(b)
            store_start(i, b)
            store_wait(b)
            gather_start(i + _NBUF, b)
        return carry

    lax.fori_loop(0, n_grp - 1, grp, 0)

    for b in range(_NBUF):
        i = (n_grp - 1) * _NBUF + b
        gather_wait(b)
        store_start(i, b)
    for b in range(_NBUF):
        store_wait(b)


def kernel(word_indexes, W):
    B, L = word_indexes.shape
    V, D = W.shape
    n = B * L
    idx = word_indexes.reshape(n).astype(jnp.int32)

    per_w = n // _NW
    assert n % _NW == 0 and per_w % (_CHUNK * _NBUF) == 0

    mesh = plsc.VectorSubcoreMesh(core_axis_name="c", subcore_axis_name="s")
    scratch = [pltpu.VMEM((per_w,), jnp.int32)]
    scratch += [pltpu.VMEM((_CHUNK, D), jnp.float32) for _ in range(_NBUF)]
    scratch += [pltpu.SemaphoreType.DMA for _ in range(2 * _NBUF)]
    k = pl.kernel(
        functools.partial(_gather_kernel, n),
        mesh=mesh,
        out_type=jax.ShapeDtypeStruct((n, D), jnp.float32),
        scratch_types=scratch,
        compiler_params=pltpu.CompilerParams(use_tc_tiling_on_sc=False),
    )
    out = k(idx, W)
    return out.reshape(B, L, D)


# speedup vs baseline: 1.9688x; 1.7686x over previous
import functools

import jax
import jax.numpy as jnp
from jax import lax
from jax.experimental import pallas as pl
from jax.experimental.pallas import tpu as pltpu
from jax.experimental.pallas import tpu_sc as plsc

_info = plsc.get_sparse_core_info()
_NC, _NS = _info.num_cores, _info.num_subcores
_NW = _NC * _NS

_NB = 4


def _gather_kernel(B, L, D, idx_hbm, table_hbm, out_hbm, idx_v,
                   rows0, rows1, g0, g1, s0, s1):
    per_w = B // _NW
    n_ch = per_w // _NB
    wid = lax.axis_index("s") * _NC + lax.axis_index("c")
    base = wid * per_w

    rows = (rows0, rows1)
    gsem = (g0, g1)
    ssem = (s0, s1)

    pltpu.sync_copy(idx_hbm.at[pl.ds(base * L, per_w * L)], idx_v)

    blocks = [(o, 0, 16) for o in range(0, L - 15, 16)]
    if L % 16:
        blocks.append((L - 16, 16 - L % 16, 16))

    def issue_chunk(c, p):
        for qq in range(_NB):
            q = c * _NB + qq
            for (o, j_lo, j_hi) in blocks:
                v = idx_v[pl.ds(q * L + o, 16)]
                for j in range(j_lo, j_hi):
                    i = v[j]
                    pltpu.async_copy(
                        table_hbm.at[pl.ds(i, 1)],
                        rows[p].at[qq, pl.ds(o + j, 1)], gsem[p])

    def wait_gathers(p):
        pltpu.make_async_copy(
            out_hbm.at[pl.ds(0, _NB)], rows[p], gsem[p]).wait()

    def store_chunk(c, p):
        pltpu.async_copy(
            rows[p], out_hbm.at[pl.ds(base + c * _NB, _NB)], ssem[p])

    def wait_store(p):
        pltpu.make_async_copy(
            rows[p], out_hbm.at[pl.ds(0, _NB)], ssem[p]).wait()

    for p in range(2):
        issue_chunk(p, p)
        wait_gathers(p)
        store_chunk(p, p)

    def pair(t, carry):
        for p in range(2):
            c = 2 * t + p
            wait_store(p)
            issue_chunk(c, p)
            wait_gathers(p)
            store_chunk(c, p)
        return carry

    lax.fori_loop(1, n_ch // 2, pair, 0)
    wait_store(0)
    wait_store(1)


def kernel(word_indexes, W):
    B, L = word_indexes.shape
    V, D = W.shape
    assert B % (_NW * _NB * 2) == 0

    idx = word_indexes.reshape(B * L).astype(jnp.int32)
    mesh = plsc.VectorSubcoreMesh(core_axis_name="c", subcore_axis_name="s")
    k = pl.kernel(
        functools.partial(_gather_kernel, B, L, D),
        mesh=mesh,
        out_type=jax.ShapeDtypeStruct((B, L, D), jnp.float32),
        scratch_types=[
            pltpu.VMEM((B * L // _NW,), jnp.int32),
            pltpu.VMEM((_NB, L, D), jnp.float32),
            pltpu.VMEM((_NB, L, D), jnp.float32),
            pltpu.SemaphoreType.DMA,
            pltpu.SemaphoreType.DMA,
            pltpu.SemaphoreType.DMA,
            pltpu.SemaphoreType.DMA,
        ],
        compiler_params=pltpu.CompilerParams(use_tc_tiling_on_sc=True),
    )
    return k(idx, W)


# speedup vs baseline: 2.0492x; 1.0408x over previous
import functools

import jax
import jax.numpy as jnp
from jax import lax
from jax.experimental import pallas as pl
from jax.experimental.pallas import tpu as pltpu
from jax.experimental.pallas import tpu_sc as plsc

_info = plsc.get_sparse_core_info()
_NC, _NS = _info.num_cores, _info.num_subcores
_NW = _NC * _NS

_NB = 4


def _gather_kernel(B, L, D, idx_hbm, table_hbm, out_hbm, idx_v,
                   rows0, rows1, g0, g1, s0, s1):
    per_w = B // _NW
    n_ch = per_w // _NB
    wid = lax.axis_index("s") * _NC + lax.axis_index("c")
    base = wid * per_w

    rows = (rows0, rows1)
    gsem = (g0, g1)
    ssem = (s0, s1)

    pltpu.sync_copy(idx_hbm.at[pl.ds(base * L, per_w * L)], idx_v)

    blocks = [(o, 0, 16) for o in range(0, L - 15, 16)]
    if L % 16:
        blocks.append((L - 16, 16 - L % 16, 16))

    def issue_chunk(c, p):
        for qq in range(_NB):
            q = c * _NB + qq
            for (o, j_lo, j_hi) in blocks:
                v = idx_v[pl.ds(q * L + o, 16)]
                for j in range(j_lo, j_hi):
                    i = v[j]
                    pltpu.async_copy(
                        table_hbm.at[pl.ds(i, 1)],
                        rows[p].at[qq, pl.ds(o + j, 1)], gsem[p])

    def wait_gathers(p):
        pltpu.make_async_copy(
            out_hbm.at[pl.ds(0, _NB)], rows[p], gsem[p]).wait()

    def store_chunk(c, p):
        pltpu.async_copy(
            rows[p], out_hbm.at[pl.ds(base + c * _NB, _NB)], ssem[p])

    def wait_store(p):
        pltpu.make_async_copy(
            rows[p], out_hbm.at[pl.ds(0, _NB)], ssem[p]).wait()

    issue_chunk(0, 0)
    issue_chunk(1, 1)

    def body(c, p):
        wait_gathers(p)
        store_chunk(c - 2, p)
        wait_store(p)
        issue_chunk(c, p)

    def pair(t, carry):
        body(2 * t, 0)
        body(2 * t + 1, 1)
        return carry

    lax.fori_loop(1, n_ch // 2, pair, 0)
    wait_gathers(0)
    store_chunk(n_ch - 2, 0)
    wait_gathers(1)
    store_chunk(n_ch - 1, 1)
    wait_store(0)
    wait_store(1)


def kernel(word_indexes, W):
    B, L = word_indexes.shape
    V, D = W.shape
    assert B % (_NW * _NB * 2) == 0

    idx = word_indexes.reshape(B * L).astype(jnp.int32)
    mesh = plsc.VectorSubcoreMesh(core_axis_name="c", subcore_axis_name="s")
    k = pl.kernel(
        functools.partial(_gather_kernel, B, L, D),
        mesh=mesh,
        out_type=jax.ShapeDtypeStruct((B, L, D), jnp.float32),
        scratch_types=[
            pltpu.VMEM((B * L // _NW,), jnp.int32),
            pltpu.VMEM((_NB, L, D), jnp.float32),
            pltpu.VMEM((_NB, L, D), jnp.float32),
            pltpu.SemaphoreType.DMA,
            pltpu.SemaphoreType.DMA,
            pltpu.SemaphoreType.DMA,
            pltpu.SemaphoreType.DMA,
        ],
        compiler_params=pltpu.CompilerParams(use_tc_tiling_on_sc=True),
    )
    return k(idx, W)


# speedup vs baseline: 2.0514x; 1.0011x over previous
import functools

import jax
import jax.numpy as jnp
from jax import lax
from jax.experimental import pallas as pl
from jax.experimental.pallas import tpu as pltpu
from jax.experimental.pallas import tpu_sc as plsc

_info = plsc.get_sparse_core_info()
_NC, _NS = _info.num_cores, _info.num_subcores
_NW = _NC * _NS

_NB = 4


def _gather_kernel(B, L, D, idx_hbm, table_hbm, out_hbm,
                   rows0, rows1, iv0, iv1, g0, g1, s0, s1, is0, is1):
    per_w = B // _NW
    n_ch = per_w // _NB
    cl = _NB * L
    wid = lax.axis_index("s") * _NC + lax.axis_index("c")
    base = wid * per_w

    rows = (rows0, rows1)
    ivec = (iv0, iv1)
    gsem = (g0, g1)
    ssem = (s0, s1)
    isem = (is0, is1)

    blocks = [(o, 0, 16) for o in range(0, cl - 15, 16)]
    if cl % 16:
        blocks.append((cl - 16, 16 - cl % 16, 16))

    def idx_start(c, p):
        pltpu.async_copy(
            idx_hbm.at[pl.ds(base * L + c * cl, cl)], ivec[p], isem[p])

    def idx_wait(p):
        pltpu.make_async_copy(
            idx_hbm.at[pl.ds(0, cl)], ivec[p], isem[p]).wait()

    def issue_chunk(c, p):
        for (o, j_lo, j_hi) in blocks:
            v = ivec[p][pl.ds(o, 16)]
            for j in range(j_lo, j_hi):
                i = v[j]
                r = o + j
                pltpu.async_copy(
                    table_hbm.at[pl.ds(i, 1)],
                    rows[p].at[r // L, pl.ds(r % L, 1)], gsem[p])

    def wait_gathers(p):
        pltpu.make_async_copy(
            out_hbm.at[pl.ds(0, _NB)], rows[p], gsem[p]).wait()

    def store_chunk(c, p):
        pltpu.async_copy(
            rows[p], out_hbm.at[pl.ds(base + c * _NB, _NB)], ssem[p])

    def wait_store(p):
        pltpu.make_async_copy(
            rows[p], out_hbm.at[pl.ds(0, _NB)], ssem[p]).wait()

    idx_start(0, 0)
    idx_start(1, 1)
    idx_wait(0)
    issue_chunk(0, 0)
    idx_start(2, 0)
    idx_wait(1)
    issue_chunk(1, 1)
    idx_start(3, 1)

    def body(c, p):
        wait_gathers(p)
        store_chunk(c - 2, p)
        wait_store(p)
        idx_wait(p)
        issue_chunk(c, p)
        idx_start(jnp.minimum(c + 2, n_ch - 1), p)

    def pair(t, carry):
        body(2 * t, 0)
        body(2 * t + 1, 1)
        return carry

    lax.fori_loop(1, n_ch // 2, pair, 0)
    wait_gathers(0)
    store_chunk(n_ch - 2, 0)
    wait_gathers(1)
    store_chunk(n_ch - 1, 1)
    idx_wait(0)
    idx_wait(1)
    wait_store(0)
    wait_store(1)


def kernel(word_indexes, W):
    B, L = word_indexes.shape
    V, D = W.shape
    assert B % (_NW * _NB * 2) == 0

    idx = word_indexes.reshape(B * L).astype(jnp.int32)
    mesh = plsc.VectorSubcoreMesh(core_axis_name="c", subcore_axis_name="s")
    k = pl.kernel(
        functools.partial(_gather_kernel, B, L, D),
        mesh=mesh,
        out_type=jax.ShapeDtypeStruct((B, L, D), jnp.float32),
        scratch_types=[
            pltpu.VMEM((_NB, L, D), jnp.float32),
            pltpu.VMEM((_NB, L, D), jnp.float32),
            pltpu.VMEM((_NB * L,), jnp.int32),
            pltpu.VMEM((_NB * L,), jnp.int32),
            pltpu.SemaphoreType.DMA,
            pltpu.SemaphoreType.DMA,
            pltpu.SemaphoreType.DMA,
            pltpu.SemaphoreType.DMA,
            pltpu.SemaphoreType.DMA,
            pltpu.SemaphoreType.DMA,
        ],
        compiler_params=pltpu.CompilerParams(use_tc_tiling_on_sc=True),
    )
    return k(idx, W)
